# baseline (speedup 1.0000x reference)
import functools

import jax
import jax.numpy as jnp
from jax import lax
from jax.experimental import pallas as pl
from jax.experimental.pallas import tpu as pltpu
from jax.experimental.pallas import tpu_sc as plsc

N = 10000
E = 160000
D = 256
HD = 128
NPC = N // 2
NT = 16
ET = E // NT
CHUNK = 80
EB = 2000
NCH = EB // CHUNK
TCH = ET // CHUNK
DUMP = 2 * NPC
ACC_ROWS = 2 * NPC + 64
ZSH = 624
ZC = 104

BM = 1000
NB = N // BM


def _mm_body(x_ref, w_ref, g_ref):
    g_ref[...] = jnp.dot(x_ref[...], w_ref[0],
                         preferred_element_type=jnp.float32)


def _tc_transform(x, wall):
    return pl.pallas_call(
        _mm_body,
        grid=(6, NB),
        in_specs=[
            pl.BlockSpec((BM, D), lambda q, i: (i, 0)),
            pl.BlockSpec((1, D, HD), lambda q, i: (q % 3, 0, q // 3)),
        ],
        out_specs=pl.BlockSpec((BM, HD), lambda q, i: (q * NB + i, 0)),
        out_shape=jax.ShapeDtypeStruct((6 * N, HD), jnp.float32),
    )(x, wall)


def _comb_body(g_ref, s0_ref, s1_ref, c0_ref, c1_ref, b_ref, o_ref):
    h = pl.program_id(0)
    c0 = jnp.maximum(c0_ref[:, 0:1], 1.0)
    c1 = jnp.maximum(c1_ref[:, 0:1], 1.0)
    o_ref[...] = (g_ref[...] + b_ref[pl.ds(h, 1), :]
                  + s0_ref[...] / c0 + s1_ref[...] / c1)


def _tc_combine(g, ss, bias2):
    return pl.pallas_call(
        _comb_body,
        grid=(2, NB),
        in_specs=[
            pl.BlockSpec((BM, HD), lambda h, i: (h * 3 * NB + i, 0)),
            pl.BlockSpec((BM, HD), lambda h, i: (h * 2 * NB + i, 0)),
            pl.BlockSpec((BM, HD), lambda h, i: ((h * 2 + 1) * NB + i, 0)),
            pl.BlockSpec((BM, HD), lambda h, i: (4 * NB + i, 0)),
            pl.BlockSpec((BM, HD), lambda h, i: (5 * NB + i, 0)),
            pl.BlockSpec((2, HD), lambda h, i: (0, 0)),
        ],
        out_specs=pl.BlockSpec((BM, HD), lambda h, i: (i, h)),
        out_shape=jax.ShapeDtypeStruct((N, D), jnp.float32),
    )(g, ss, ss, ss, ss, bias2)


def _sc_body(g_hbm, src_hbm, dst_hbm, et_hbm,
             s_hbm,
             acc,
             src_v, dst_v, et_v,
             rows_0, rows_1, rows_2, rows_3,
             gidx_0, gidx_1, gidx_2, gidx_3,
             sidx_0, sidx_1, sidx_2, sidx_3,
             gsem_0, gsem_1, gsem_2, gsem_3,
             ssem_0, ssem_1, ssem_2, ssem_3):
    c = lax.axis_index("c")
    s = lax.axis_index("s")
    lo = c * NPC

    rows = (rows_0, rows_1, rows_2, rows_3)
    gidx = (gidx_0, gidx_1, gidx_2, gidx_3)
    sidx = (sidx_0, sidx_1, sidx_2, sidx_3)
    gsem = (gsem_0, gsem_1, gsem_2, gsem_3)
    ssem = (ssem_0, ssem_1, ssem_2, ssem_3)

    zeros16 = jnp.zeros((16,), jnp.float32)
    ones16 = jnp.ones((16,), jnp.float32)

    def fill_rows0(val):
        def body(i, carry):
            for j in range(HD // 16):
                rows_0[i, pl.ds(j * 16, 16)] = val
            return carry

        lax.fori_loop(0, CHUNK, body, 0)

    def fill_ones(i, carry):
        for j in range(HD // 16):
            rows_1[i, pl.ds(j * 16, 16)] = ones16
            rows_2[i, pl.ds(j * 16, 16)] = ones16
            rows_3[i, pl.ds(j * 16, 16)] = ones16
        return carry

    zbase = s * ZSH
    r = s // 8
    m = s % 8
    wrow0 = r * NPC + m * ZSH

    def zero_acc():
        fill_rows0(zeros16)

        def zb(k, carry):
            dst = pl.multiple_of(zbase + k * CHUNK, 8)
            pltpu.sync_copy(rows_0, acc.at[pl.ds(dst, CHUNK)])
            return carry

        def ztail64(k, carry):
            dst = pl.multiple_of(zbase + 7 * CHUNK, 8)
            pltpu.sync_copy(rows_0.at[pl.ds(0, 64)], acc.at[pl.ds(dst, 64)])
            return carry

        def ztail64b(k, carry):
            dst = pl.multiple_of(15 * ZSH + 8 * CHUNK, 8)
            pltpu.sync_copy(rows_0.at[pl.ds(0, 64)], acc.at[pl.ds(dst, 64)])
            return carry

        lax.fori_loop(0, jnp.where(s < 15, 7, 8), zb, 0)
        lax.fori_loop(0, jnp.where(s < 15, 1, 0), ztail64, 0)
        lax.fori_loop(0, jnp.where(s == 15, 1, 0), ztail64b, 0)

    def run_pass(h):
        h3n = h * 3 * N
        gate = jnp.where(h < 2, 1, 0)

        def stage_if(q):
            def body(k, carry):
                e_base = pl.multiple_of(s * ET + (q // NCH) * EB, 8)
                pltpu.sync_copy(src_hbm.at[pl.ds(e_base, EB)], src_v)
                pltpu.sync_copy(dst_hbm.at[pl.ds(e_base, EB)], dst_v)
                pltpu.sync_copy(et_hbm.at[pl.ds(e_base, EB)], et_v)
                return carry

            lax.fori_loop(0, jnp.where(q % NCH == 0, 1, 0), body, 0)

        def keys(q, j):
            e0 = (q % NCH) * CHUNK
            for g in range(CHUNK // 16):
                d16 = dst_v[pl.ds(e0 + g * 16, 16)]
                t16 = et_v[pl.ds(e0 + g * 16, 16)]
                own = (d16 >= lo) & (d16 < lo + NPC)
                sidx[j][pl.ds(g * 16, 16)] = jnp.where(
                    own, t16 * NPC + (d16 - lo), DUMP + (d16 & 63))
                s16 = src_v[pl.ds(e0 + g * 16, 16)]
                gidx[j][pl.ds(g * 16, 16)] = (t16 + 1) * N + s16 + h3n

        def gstart(j):
            def body(k, carry):
                pltpu.async_copy(g_hbm.at[gidx[j]], rows[j], gsem[j])
                return carry

            lax.fori_loop(0, gate, body, 0)

        def gwait(j):
            def body(k, carry):
                pltpu.make_async_copy(g_hbm.at[gidx[j]], rows[j],
                                      gsem[j]).wait()
                return carry

            lax.fori_loop(0, gate, body, 0)

        def sstart(j):
            pltpu.async_copy(rows[j], acc.at[sidx[j]], ssem[j], add=True)

        def swait(j, trip):
            def body(k, carry):
                pltpu.make_async_copy(rows[j], acc.at[sidx[j]],
                                      ssem[j]).wait()
                return carry

            lax.fori_loop(0, trip, body, 0)

        def issue(q, j, k):
            swait(j, jnp.where(q >= 4, 1, 0))
            stage_if(q)
            keys(q, j)
            gstart(j)

        def drain(q, j):
            gwait(j)
            sstart(j)

        def quad(k, carry):
            for j in range(4):
                q = 4 * k + j
                issue(q, j, k)
                jd = (j + 2) % 4

                def do_drain(_, carry2):
                    drain(q - 2, jd)
                    return carry2

                lax.fori_loop(0, jnp.where(q >= 2, 1, 0), do_drain, 0)
            return carry

        lax.fori_loop(0, TCH // 4, quad, 0)
        issue(TCH - 1, 0, 0)
        drain(TCH - 3, 2)
        drain(TCH - 2, 3)
        drain(TCH - 1, 0)
        swait(0, 1)
        swait(1, 1)
        swait(2, 1)
        swait(3, 1)

    def writeback(h):
        srow0 = h * 2 * N + r * N + lo + m * ZSH

        def wb(k, carry):
            a = pl.multiple_of(wrow0 + k * ZC, 8)
            pltpu.sync_copy(acc.at[pl.ds(a, ZC)],
                            s_hbm.at[pl.ds(pl.multiple_of(srow0 + k * ZC, 8),
                                           ZC)])
            return carry

        def wtail(k, carry):
            a = pl.multiple_of(wrow0 + 6 * ZC, 8)
            pltpu.sync_copy(
                acc.at[pl.ds(a, 8)],
                s_hbm.at[pl.ds(pl.multiple_of(srow0 + 6 * ZC, 8), 8)])
            return carry

        lax.fori_loop(0, ZSH // ZC, wb, 0)
        lax.fori_loop(0, jnp.where(m == 7, 1, 0), wtail, 0)

    def h_pass(h, carry):
        zero_acc()
        lax.fori_loop(0, jnp.where(h == 2, CHUNK, 0), fill_ones, 0)
        lax.fori_loop(0, jnp.where(h == 2, 1, 0),
                      lambda k, cr: (fill_rows0(ones16), cr)[1], 0)
        plsc.subcore_barrier()
        run_pass(h)
        plsc.subcore_barrier()
        writeback(h)
        plsc.subcore_barrier()
        return carry

    lax.fori_loop(0, 3, h_pass, 0)


@functools.cache
def _make_sc_scatter():
    return pl.kernel(
        _sc_body,
        out_type=jax.ShapeDtypeStruct((6 * N, HD), jnp.float32),
        mesh=plsc.VectorSubcoreMesh(core_axis_name="c",
                                    subcore_axis_name="s"),
        scratch_types=(
            pltpu.VMEM_SHARED((ACC_ROWS, HD), jnp.float32),
            pltpu.VMEM((EB,), jnp.int32),
            pltpu.VMEM((EB,), jnp.int32),
            pltpu.VMEM((EB,), jnp.int32),
            pltpu.VMEM((CHUNK, HD), jnp.float32),
            pltpu.VMEM((CHUNK, HD), jnp.float32),
            pltpu.VMEM((CHUNK, HD), jnp.float32),
            pltpu.VMEM((CHUNK, HD), jnp.float32),
            pltpu.VMEM((CHUNK,), jnp.int32),
            pltpu.VMEM((CHUNK,), jnp.int32),
            pltpu.VMEM((CHUNK,), jnp.int32),
            pltpu.VMEM((CHUNK,), jnp.int32),
            pltpu.VMEM((CHUNK,), jnp.int32),
            pltpu.VMEM((CHUNK,), jnp.int32),
            pltpu.VMEM((CHUNK,), jnp.int32),
            pltpu.VMEM((CHUNK,), jnp.int32),
            pltpu.SemaphoreType.DMA,
            pltpu.SemaphoreType.DMA,
            pltpu.SemaphoreType.DMA,
            pltpu.SemaphoreType.DMA,
            pltpu.SemaphoreType.DMA,
            pltpu.SemaphoreType.DMA,
            pltpu.SemaphoreType.DMA,
            pltpu.SemaphoreType.DMA,
        ),
    )


def kernel(x, edge_index, edge_type, W, W_root, bias):
    wall = jnp.concatenate([W_root[None], W], axis=0)
    g = _tc_transform(x, wall)
    src = edge_index[0].astype(jnp.int32)
    dst = edge_index[1].astype(jnp.int32)
    et = edge_type.astype(jnp.int32)
    ss = _make_sc_scatter()(g, src, dst, et)
    bias2 = bias.reshape(2, HD)
    return _tc_combine(g, ss, bias2)


# speedup vs baseline: 1.0107x; 1.0107x over previous
import functools

import jax
import jax.numpy as jnp
from jax import lax
from jax.experimental import pallas as pl
from jax.experimental.pallas import tpu as pltpu
from jax.experimental.pallas import tpu_sc as plsc

N = 10000
E = 160000
D = 256
HD = 128
NPC = N // 2
NT = 16
ET = E // NT
CHUNK = 80
EB = 2000
NCH = EB // CHUNK
TCH = ET // CHUNK
DUMP = 2 * NPC
ACC_ROWS = 2 * NPC + 64
ZSH = 624
ZC = 104

BM = 1000
NB = N // BM


def _mm_body(x_ref, w_ref, g_ref):
    g_ref[...] = jnp.dot(x_ref[...], w_ref[0],
                         preferred_element_type=jnp.float32)


def _tc_transform(x, wall):
    return pl.pallas_call(
        _mm_body,
        grid=(6, NB),
        in_specs=[
            pl.BlockSpec((BM, D), lambda q, i: (i, 0)),
            pl.BlockSpec((1, D, HD), lambda q, i: (q % 3, 0, q // 3)),
        ],
        out_specs=pl.BlockSpec((BM, HD), lambda q, i: (q * NB + i, 0)),
        out_shape=jax.ShapeDtypeStruct((6 * N, HD), jnp.float32),
    )(x, wall)


def _comb_body(g_ref, s0_ref, s1_ref, c0_ref, c1_ref, b_ref, o_ref):
    h = pl.program_id(0)
    c0 = jnp.maximum(c0_ref[:, 0:1], 1.0)
    c1 = jnp.maximum(c1_ref[:, 0:1], 1.0)
    o_ref[...] = (g_ref[...] + b_ref[pl.ds(h, 1), :]
                  + s0_ref[...] / c0 + s1_ref[...] / c1)


def _tc_combine(g, ss, bias2):
    return pl.pallas_call(
        _comb_body,
        grid=(2, NB),
        in_specs=[
            pl.BlockSpec((BM, HD), lambda h, i: (h * 3 * NB + i, 0)),
            pl.BlockSpec((BM, HD), lambda h, i: (h * 2 * NB + i, 0)),
            pl.BlockSpec((BM, HD), lambda h, i: ((h * 2 + 1) * NB + i, 0)),
            pl.BlockSpec((BM, HD), lambda h, i: (4 * NB + i, 0)),
            pl.BlockSpec((BM, HD), lambda h, i: (5 * NB + i, 0)),
            pl.BlockSpec((2, HD), lambda h, i: (0, 0)),
        ],
        out_specs=pl.BlockSpec((BM, HD), lambda h, i: (i, h)),
        out_shape=jax.ShapeDtypeStruct((N, D), jnp.float32),
    )(g, ss, ss, ss, ss, bias2)


def _sc_body(g_hbm, src_hbm, dst_hbm, et_hbm,
             s_hbm,
             acc,
             src_v, dst_v, et_v,
             rows_0, rows_1, rows_2, rows_3,
             gidx_0, gidx_1, gidx_2, gidx_3,
             sidx_0, sidx_1, sidx_2, sidx_3,
             gsem_0, gsem_1, gsem_2, gsem_3,
             ssem_0, ssem_1, ssem_2, ssem_3):
    c = lax.axis_index("c")
    s = lax.axis_index("s")
    lo = c * NPC

    rows = (rows_0, rows_1, rows_2, rows_3)
    gidx = (gidx_0, gidx_1, gidx_2, gidx_3)
    sidx = (sidx_0, sidx_1, sidx_2, sidx_3)
    gsem = (gsem_0, gsem_1, gsem_2, gsem_3)
    ssem = (ssem_0, ssem_1, ssem_2, ssem_3)

    zeros16 = jnp.zeros((16,), jnp.float32)
    ones16 = jnp.ones((16,), jnp.float32)

    def fill_rows0(val):
        def body(i, carry):
            for j in range(HD // 16):
                rows_0[i, pl.ds(j * 16, 16)] = val
            return carry

        lax.fori_loop(0, CHUNK, body, 0)

    def fill_ones(i, carry):
        for j in range(HD // 16):
            rows_1[i, pl.ds(j * 16, 16)] = ones16
            rows_2[i, pl.ds(j * 16, 16)] = ones16
            rows_3[i, pl.ds(j * 16, 16)] = ones16
        return carry

    zbase = s * ZSH
    r = s // 8
    m = s % 8
    wrow0 = r * NPC + m * ZSH

    def zero_acc():
        fill_rows0(zeros16)

        def zb(k, carry):
            dst = pl.multiple_of(zbase + k * CHUNK, 8)
            pltpu.sync_copy(rows_0, acc.at[pl.ds(dst, CHUNK)])
            return carry

        def ztail64(k, carry):
            dst = pl.multiple_of(zbase + 7 * CHUNK, 8)
            pltpu.sync_copy(rows_0.at[pl.ds(0, 64)], acc.at[pl.ds(dst, 64)])
            return carry

        def ztail64b(k, carry):
            dst = pl.multiple_of(15 * ZSH + 8 * CHUNK, 8)
            pltpu.sync_copy(rows_0.at[pl.ds(0, 64)], acc.at[pl.ds(dst, 64)])
            return carry

        lax.fori_loop(0, jnp.where(s < 15, 7, 8), zb, 0)
        lax.fori_loop(0, jnp.where(s < 15, 1, 0), ztail64, 0)
        lax.fori_loop(0, jnp.where(s == 15, 1, 0), ztail64b, 0)

    def run_pass(h):
        h3n = h * 3 * N
        gate = jnp.where(h < 2, 1, 0)

        def stage_if(q):
            def body(k, carry):
                e_base = pl.multiple_of(s * ET + (q // NCH) * EB, 8)
                pltpu.sync_copy(src_hbm.at[pl.ds(e_base, EB)], src_v)
                pltpu.sync_copy(dst_hbm.at[pl.ds(e_base, EB)], dst_v)
                pltpu.sync_copy(et_hbm.at[pl.ds(e_base, EB)], et_v)
                return carry

            lax.fori_loop(0, jnp.where(q % NCH == 0, 1, 0), body, 0)

        def keys(q, j):
            e0 = (q % NCH) * CHUNK
            for g in range(CHUNK // 16):
                d16 = dst_v[pl.ds(e0 + g * 16, 16)]
                t16 = et_v[pl.ds(e0 + g * 16, 16)]
                own = (d16 >= lo) & (d16 < lo + NPC)
                sidx[j][pl.ds(g * 16, 16)] = jnp.where(
                    own, t16 * NPC + (d16 - lo), DUMP + (d16 & 63))
                s16 = src_v[pl.ds(e0 + g * 16, 16)]
                gidx[j][pl.ds(g * 16, 16)] = (t16 + 1) * N + s16 + h3n

        def gstart(j):
            def body(k, carry):
                pltpu.async_copy(g_hbm.at[gidx[j]], rows[j], gsem[j])
                return carry

            lax.fori_loop(0, gate, body, 0)

        def gwait(j):
            def body(k, carry):
                pltpu.make_async_copy(g_hbm.at[gidx[j]], rows[j],
                                      gsem[j]).wait()
                return carry

            lax.fori_loop(0, gate, body, 0)

        def sstart(j):
            pltpu.async_copy(rows[j], acc.at[sidx[j]], ssem[j], add=True)

        def swait(j, trip):
            def body(k, carry):
                pltpu.make_async_copy(rows[j], acc.at[sidx[j]],
                                      ssem[j]).wait()
                return carry

            lax.fori_loop(0, trip, body, 0)

        def issue(q, j, k):
            swait(j, jnp.where(q >= 4, 1, 0))
            stage_if(q)
            keys(q, j)
            gstart(j)

        def drain(q, j):
            gwait(j)
            sstart(j)

        def quad(k, carry):
            for j in range(4):
                q = 4 * k + j
                issue(q, j, k)
                jd = (j + 1) % 4

                def do_drain(_, carry2):
                    drain(q - 3, jd)
                    return carry2

                lax.fori_loop(0, jnp.where(q >= 3, 1, 0), do_drain, 0)
            return carry

        lax.fori_loop(0, TCH // 4, quad, 0)
        issue(TCH - 1, 0, 0)
        drain(TCH - 4, 1)
        drain(TCH - 3, 2)
        drain(TCH - 2, 3)
        drain(TCH - 1, 0)
        swait(0, 1)
        swait(1, 1)
        swait(2, 1)
        swait(3, 1)

    def writeback(h):
        srow0 = h * 2 * N + r * N + lo + m * ZSH

        def wb(k, carry):
            a = pl.multiple_of(wrow0 + k * ZC, 8)
            pltpu.sync_copy(acc.at[pl.ds(a, ZC)],
                            s_hbm.at[pl.ds(pl.multiple_of(srow0 + k * ZC, 8),
                                           ZC)])
            return carry

        def wtail(k, carry):
            a = pl.multiple_of(wrow0 + 6 * ZC, 8)
            pltpu.sync_copy(
                acc.at[pl.ds(a, 8)],
                s_hbm.at[pl.ds(pl.multiple_of(srow0 + 6 * ZC, 8), 8)])
            return carry

        lax.fori_loop(0, ZSH // ZC, wb, 0)
        lax.fori_loop(0, jnp.where(m == 7, 1, 0), wtail, 0)

    def h_pass(h, carry):
        zero_acc()
        lax.fori_loop(0, jnp.where(h == 2, CHUNK, 0), fill_ones, 0)
        lax.fori_loop(0, jnp.where(h == 2, 1, 0),
                      lambda k, cr: (fill_rows0(ones16), cr)[1], 0)
        plsc.subcore_barrier()
        run_pass(h)
        plsc.subcore_barrier()
        writeback(h)
        plsc.subcore_barrier()
        return carry

    lax.fori_loop(0, 3, h_pass, 0)


@functools.cache
def _make_sc_scatter():
    return pl.kernel(
        _sc_body,
        out_type=jax.ShapeDtypeStruct((6 * N, HD), jnp.float32),
        mesh=plsc.VectorSubcoreMesh(core_axis_name="c",
                                    subcore_axis_name="s"),
        scratch_types=(
            pltpu.VMEM_SHARED((ACC_ROWS, HD), jnp.float32),
            pltpu.VMEM((EB,), jnp.int32),
            pltpu.VMEM((EB,), jnp.int32),
            pltpu.VMEM((EB,), jnp.int32),
            pltpu.VMEM((CHUNK, HD), jnp.float32),
            pltpu.VMEM((CHUNK, HD), jnp.float32),
            pltpu.VMEM((CHUNK, HD), jnp.float32),
            pltpu.VMEM((CHUNK, HD), jnp.float32),
            pltpu.VMEM((CHUNK,), jnp.int32),
            pltpu.VMEM((CHUNK,), jnp.int32),
            pltpu.VMEM((CHUNK,), jnp.int32),
            pltpu.VMEM((CHUNK,), jnp.int32),
            pltpu.VMEM((CHUNK,), jnp.int32),
            pltpu.VMEM((CHUNK,), jnp.int32),
            pltpu.VMEM((CHUNK,), jnp.int32),
            pltpu.VMEM((CHUNK,), jnp.int32),
            pltpu.SemaphoreType.DMA,
            pltpu.SemaphoreType.DMA,
            pltpu.SemaphoreType.DMA,
            pltpu.SemaphoreType.DMA,
            pltpu.SemaphoreType.DMA,
            pltpu.SemaphoreType.DMA,
            pltpu.SemaphoreType.DMA,
            pltpu.SemaphoreType.DMA,
        ),
    )


def kernel(x, edge_index, edge_type, W, W_root, bias):
    wall = jnp.concatenate([W_root[None], W], axis=0)
    g = _tc_transform(x, wall)
    src = edge_index[0].astype(jnp.int32)
    dst = edge_index[1].astype(jnp.int32)
    et = edge_type.astype(jnp.int32)
    ss = _make_sc_scatter()(g, src, dst, et)
    bias2 = bias.reshape(2, HD)
    return _tc_combine(g, ss, bias2)
